# Initial kernel scaffold; baseline (speedup 1.0000x reference)
import jax
import jax.numpy as jnp
from jax.experimental import pallas as pl


def kernel(story, table):
    raise NotImplementedError("write your pallas kernel here")


# speedup vs baseline: 2.2002x; 2.2002x over previous
import functools

import jax
import jax.numpy as jnp
from jax import lax
from jax.experimental import pallas as pl
from jax.experimental.pallas import tpu as pltpu
from jax.experimental.pallas import tpu_sc as plsc

EMB = 64
SEQ = 200
SEQP = 208
BATCH = 16384
LANES = 16
CH = 8
NIDX = CH * SEQP
GCHUNK = 128
NGATHER = NIDX // GCHUNK
NC = 2
NS = 16
NW = NC * NS
ROWS_PER_W = BATCH // NW
NCHUNKS = ROWS_PER_W // CH


def _body(story_hbm, table_hbm, out_hbm, story_v, idx_v, rows_v, sem):
    wid = lax.axis_index("s") * NC + lax.axis_index("c")
    base_row = wid * ROWS_PER_W

    def chunk_body(g, _):
        row0 = base_row + g * CH
        pltpu.sync_copy(story_hbm.at[pl.ds(row0, CH), :], story_v)

        for r in range(CH):
            carry = jnp.int32(0)
            for j in range(SEQP // LANES):
                tok = story_v[r, pl.ds(j * LANES, LANES)]
                m = tok != 0
                ones = jnp.where(m, jnp.int32(1), jnp.int32(0))
                csum = plsc.cumsum(ones)
                pos = jnp.where(m, csum + carry, jnp.int32(0))
                idx_v[pl.ds(r * SEQP + j * LANES, LANES)] = pos
                carry = carry + jnp.sum(ones)

        copies = []
        for j in range(NGATHER):
            copies.append(
                pltpu.async_copy(
                    table_hbm.at[idx_v.at[pl.ds(j * GCHUNK, GCHUNK)]],
                    rows_v.at[pl.ds(j * GCHUNK, GCHUNK)],
                    sem,
                )
            )
        for cp in copies:
            cp.wait()

        for r in range(CH):
            pltpu.sync_copy(
                rows_v.at[pl.ds(r * SEQP, SEQ), :],
                out_hbm.at[row0 + r],
            )
        return ()

    lax.fori_loop(0, NCHUNKS, chunk_body, ())


@functools.partial(jax.jit, donate_argnums=())
def _encode(story2, table):
    mesh = plsc.VectorSubcoreMesh(core_axis_name="c", subcore_axis_name="s")
    f = functools.partial(
        pl.kernel,
        mesh=mesh,
        out_type=jax.ShapeDtypeStruct((BATCH, SEQ, EMB), jnp.float32),
        scratch_types=[
            pltpu.VMEM((CH, SEQP), jnp.int32),
            pltpu.VMEM((NIDX,), jnp.int32),
            pltpu.VMEM((NIDX, EMB), jnp.float32),
            pltpu.SemaphoreType.DMA,
        ],
        compiler_params=pltpu.CompilerParams(
            needs_layout_passes=False, use_tc_tiling_on_sc=False
        ),
    )(_body)
    return f(story2, table)


def kernel(story, table):
    story2 = jnp.pad(story[:, :, 0], ((0, 0), (0, SEQP - SEQ)))
    return _encode(story2, table)


# speedup vs baseline: 3.0756x; 1.3979x over previous
import functools

import jax
import jax.numpy as jnp
from jax import lax
from jax.experimental import pallas as pl
from jax.experimental.pallas import tpu as pltpu
from jax.experimental.pallas import tpu_sc as plsc

EMB = 64
SEQ = 200
BATCH = 16384
LANES = 16
CH = 4
NIDX = CH * SEQ
GCHUNKS = [(0, 128), (128, 128), (256, 128), (384, 128), (512, 128),
           (640, 128), (768, 32)]
NC = 2
NS = 16
NW = NC * NS
ROWS_PER_W = BATCH // NW
NPAIRS = ROWS_PER_W // (2 * CH)
NFULL = SEQ // LANES
TAIL_OFF = SEQ - LANES


def _positions_row(story_v, idx_v, sr, r):
    lane = lax.iota(jnp.int32, 16)
    carry = jnp.int32(0)
    carry183 = jnp.int32(0)
    for j in range(NFULL):
        tok = story_v[sr, pl.ds(j * LANES, LANES)]
        m = tok != 0
        ones = jnp.where(m, jnp.int32(1), jnp.int32(0))
        csum = plsc.cumsum(ones)
        idx_v[pl.ds(r * SEQ + j * LANES, LANES)] = jnp.where(
            m, csum + carry, jnp.int32(0)
        )
        if j == NFULL - 1:
            carry183 = carry + jnp.sum(jnp.where(lane < 8, ones, jnp.int32(0)))
        carry = carry + jnp.sum(ones)
    tok = story_v[sr, pl.ds(TAIL_OFF, LANES)]
    m = tok != 0
    ones = jnp.where(m, jnp.int32(1), jnp.int32(0))
    csum = plsc.cumsum(ones)
    idx_v[pl.ds(r * SEQ + TAIL_OFF, LANES)] = jnp.where(
        m, csum + carry183, jnp.int32(0)
    )


def _body(story_hbm, table_hbm, out_hbm, story_v, idx0_v, idx1_v, rows0_v,
          rows1_v, sem_g0, sem_g1, sem_o):
    wid = lax.axis_index("s") * NC + lax.axis_index("c")
    base_row = wid * ROWS_PER_W

    idx_bufs = (idx0_v, idx1_v)
    rows_bufs = (rows0_v, rows1_v)
    gsems = (sem_g0, sem_g1)

    def fire_gathers(idx_v, rows_v, sem):
        return [
            pltpu.async_copy(
                table_hbm.at[idx_v.at[pl.ds(off, n)]],
                rows_v.at[pl.ds(off, n)],
                sem,
            )
            for off, n in GCHUNKS
        ]

    def fire_outs(rows_v, row0):
        return [
            pltpu.async_copy(
                rows_v.at[pl.ds(r * SEQ, SEQ), :],
                out_hbm.at[pl.ds((row0 + r) * SEQ, SEQ), :],
                sem_o,
            )
            for r in range(CH)
        ]

    def pair_body(i, _):
        row0 = base_row + i * 2 * CH
        pltpu.sync_copy(story_hbm.at[pl.ds(row0, 2 * CH), :], story_v)

        for r in range(CH):
            _positions_row(story_v, idx_bufs[0], r, r)
        g0 = fire_gathers(idx_bufs[0], rows_bufs[0], gsems[0])

        for r in range(CH):
            _positions_row(story_v, idx_bufs[1], CH + r, r)

        for cp in g0:
            cp.wait()
        o0 = fire_outs(rows_bufs[0], row0)

        g1 = fire_gathers(idx_bufs[1], rows_bufs[1], gsems[1])
        for cp in g1:
            cp.wait()
        o1 = fire_outs(rows_bufs[1], row0 + CH)

        for cp in o0:
            cp.wait()
        for cp in o1:
            cp.wait()
        return ()

    lax.fori_loop(0, NPAIRS, pair_body, ())


@jax.jit
def _encode(story2, table):
    mesh = plsc.VectorSubcoreMesh(core_axis_name="c", subcore_axis_name="s")
    f = functools.partial(
        pl.kernel,
        mesh=mesh,
        out_type=jax.ShapeDtypeStruct((BATCH * SEQ, EMB), jnp.float32),
        scratch_types=[
            pltpu.VMEM((2 * CH, SEQ), jnp.int32),
            pltpu.VMEM((NIDX,), jnp.int32),
            pltpu.VMEM((NIDX,), jnp.int32),
            pltpu.VMEM((NIDX, EMB), jnp.float32),
            pltpu.VMEM((NIDX, EMB), jnp.float32),
            pltpu.SemaphoreType.DMA,
            pltpu.SemaphoreType.DMA,
            pltpu.SemaphoreType.DMA,
        ],
        compiler_params=pltpu.CompilerParams(
            needs_layout_passes=False, use_tc_tiling_on_sc=False
        ),
    )(_body)
    return f(story2, table)


def kernel(story, table):
    out = _encode(story[:, :, 0], table)
    return out.reshape(BATCH, SEQ, EMB)


# speedup vs baseline: 4.0113x; 1.3042x over previous
import functools

import jax
import jax.numpy as jnp
from jax import lax
from jax.experimental import pallas as pl
from jax.experimental.pallas import tpu as pltpu
from jax.experimental.pallas import tpu_sc as plsc

EMB = 64
SEQ = 200
BATCH = 16384
LANES = 16
CH = 4
NIDX = CH * SEQ
G0 = 104
G1 = SEQ - G0
NC = 2
NS = 16
NW = NC * NS
ROWS_PER_W = BATCH // NW
NPAIRS = ROWS_PER_W // (2 * CH)
NFULL = SEQ // LANES
TAIL_OFF = SEQ - LANES


def _positions_row(story_v, idx_v, sr, r):
    lane = lax.iota(jnp.int32, 16)
    carry = jnp.int32(0)
    carry183 = jnp.int32(0)
    for j in range(NFULL):
        tok = story_v[sr, pl.ds(j * LANES, LANES)]
        m = tok != 0
        ones = jnp.where(m, jnp.int32(1), jnp.int32(0))
        csum = plsc.cumsum(ones)
        idx_v[pl.ds(r * SEQ + j * LANES, LANES)] = jnp.where(
            m, csum + carry, jnp.int32(0)
        )
        if j == NFULL - 1:
            carry183 = carry + jnp.sum(jnp.where(lane < 8, ones, jnp.int32(0)))
        carry = carry + jnp.sum(ones)
    tok = story_v[sr, pl.ds(TAIL_OFF, LANES)]
    m = tok != 0
    ones = jnp.where(m, jnp.int32(1), jnp.int32(0))
    csum = plsc.cumsum(ones)
    idx_v[pl.ds(r * SEQ + TAIL_OFF, LANES)] = jnp.where(
        m, csum + carry183, jnp.int32(0)
    )
    return carry183 + jnp.sum(ones)


def _body(story_hbm, table_hbm, out_hbm, story_v, idx0_v, idx1_v, rows0_v,
          rows1_v, sem_g, sem_o):
    wid = lax.axis_index("s") * NC + lax.axis_index("c")
    base_row = wid * ROWS_PER_W

    idx_bufs = (idx0_v, idx1_v)
    rows_bufs = (rows0_v, rows1_v)
    tab_rows = table_hbm.at[pl.ds(1, SEQ), :]

    fills = [
        pltpu.async_copy(tab_rows, rows.at[pl.ds(r * SEQ, SEQ), :], sem_o)
        for rows in rows_bufs
        for r in range(CH)
    ]
    for cp in fills:
        cp.wait()

    def gather_slow_rows(idx_v, rows_v, counts):
        for r in range(CH):
            @pl.when(counts[r] != SEQ)
            def _slow():
                ga = pltpu.async_copy(
                    table_hbm.at[idx_v.at[pl.ds(r * SEQ, G0)]],
                    rows_v.at[pl.ds(r * SEQ, G0)],
                    sem_g,
                )
                gb = pltpu.async_copy(
                    table_hbm.at[idx_v.at[pl.ds(r * SEQ + G0, G1)]],
                    rows_v.at[pl.ds(r * SEQ + G0, G1)],
                    sem_g,
                )
                ga.wait()
                gb.wait()

    def fire_outs(rows_v, row0):
        return [
            pltpu.async_copy(
                rows_v.at[pl.ds(r * SEQ, SEQ), :],
                out_hbm.at[pl.ds((row0 + r) * SEQ, SEQ), :],
                sem_o,
            )
            for r in range(CH)
        ]

    def refill_slow_rows(rows_v, counts):
        for r in range(CH):
            @pl.when(counts[r] != SEQ)
            def _refill():
                pltpu.async_copy(
                    tab_rows, rows_v.at[pl.ds(r * SEQ, SEQ), :], sem_g
                ).wait()

    def pair_body(i, _):
        row0 = base_row + i * 2 * CH
        pltpu.sync_copy(story_hbm.at[pl.ds(row0, 2 * CH), :], story_v)

        counts0 = [_positions_row(story_v, idx_bufs[0], r, r)
                   for r in range(CH)]
        gather_slow_rows(idx_bufs[0], rows_bufs[0], counts0)
        o0 = fire_outs(rows_bufs[0], row0)

        counts1 = [_positions_row(story_v, idx_bufs[1], CH + r, r)
                   for r in range(CH)]
        gather_slow_rows(idx_bufs[1], rows_bufs[1], counts1)
        o1 = fire_outs(rows_bufs[1], row0 + CH)

        for cp in o0:
            cp.wait()
        refill_slow_rows(rows_bufs[0], counts0)
        for cp in o1:
            cp.wait()
        refill_slow_rows(rows_bufs[1], counts1)
        return ()

    lax.fori_loop(0, NPAIRS, pair_body, ())


@jax.jit
def _encode(story2, table):
    mesh = plsc.VectorSubcoreMesh(core_axis_name="c", subcore_axis_name="s")
    f = functools.partial(
        pl.kernel,
        mesh=mesh,
        out_type=jax.ShapeDtypeStruct((BATCH * SEQ, EMB), jnp.float32),
        scratch_types=[
            pltpu.VMEM((2 * CH, SEQ), jnp.int32),
            pltpu.VMEM((NIDX,), jnp.int32),
            pltpu.VMEM((NIDX,), jnp.int32),
            pltpu.VMEM((NIDX, EMB), jnp.float32),
            pltpu.VMEM((NIDX, EMB), jnp.float32),
            pltpu.SemaphoreType.DMA,
            pltpu.SemaphoreType.DMA,
        ],
        compiler_params=pltpu.CompilerParams(
            needs_layout_passes=False, use_tc_tiling_on_sc=False
        ),
    )(_body)
    return f(story2, table)


def kernel(story, table):
    out = _encode(story[:, :, 0], table)
    return out.reshape(BATCH, SEQ, EMB)


# speedup vs baseline: 4.0186x; 1.0018x over previous
import functools

import jax
import jax.numpy as jnp
from jax import lax
from jax.experimental import pallas as pl
from jax.experimental.pallas import tpu as pltpu
from jax.experimental.pallas import tpu_sc as plsc

EMB = 64
SEQ = 200
BATCH = 16384
LANES = 16
CH = 4
NIDX = CH * SEQ
G0 = 104
G1 = SEQ - G0
NC = 2
NS = 16
NW = NC * NS
ROWS_PER_W = BATCH // NW
NPAIRS = ROWS_PER_W // (2 * CH)
NFULL = SEQ // LANES
TAIL_OFF = SEQ - LANES


def _positions_row(story_v, idx_v, sr, r):
    lane = lax.iota(jnp.int32, 16)
    low8 = lane < 8
    zeros = jnp.zeros(LANES, jnp.int32)
    carry = zeros
    carry183 = zeros
    for j in range(NFULL):
        tok = story_v[sr, pl.ds(j * LANES, LANES)]
        m = tok != 0
        ones = jnp.where(m, jnp.int32(1), jnp.int32(0))
        csum = plsc.cumsum(ones)
        idx_v[pl.ds(r * SEQ + j * LANES, LANES)] = jnp.where(
            m, csum + carry, jnp.int32(0)
        )
        if j == NFULL - 1:
            carry183 = carry + plsc.all_reduce_population_count(
                jnp.logical_and(m, low8))
        carry = carry + plsc.all_reduce_population_count(m)
    tok = story_v[sr, pl.ds(TAIL_OFF, LANES)]
    m = tok != 0
    ones = jnp.where(m, jnp.int32(1), jnp.int32(0))
    csum = plsc.cumsum(ones)
    idx_v[pl.ds(r * SEQ + TAIL_OFF, LANES)] = jnp.where(
        m, csum + carry183, jnp.int32(0)
    )
    total = carry183 + plsc.all_reduce_population_count(m)
    return lax.reduce_max(total, axes=(0,))


def _body(story_hbm, table_hbm, out_hbm, story_v, idx0_v, idx1_v, rows0_v,
          rows1_v, sem_g, sem_o):
    wid = lax.axis_index("s") * NC + lax.axis_index("c")
    base_row = wid * ROWS_PER_W

    idx_bufs = (idx0_v, idx1_v)
    rows_bufs = (rows0_v, rows1_v)
    tab_rows = table_hbm.at[pl.ds(1, SEQ), :]

    fills = [
        pltpu.async_copy(tab_rows, rows.at[pl.ds(r * SEQ, SEQ), :], sem_o)
        for rows in rows_bufs
        for r in range(CH)
    ]
    for cp in fills:
        cp.wait()

    def gather_slow_rows(idx_v, rows_v, counts):
        for r in range(CH):
            @pl.when(counts[r] != SEQ)
            def _slow():
                ga = pltpu.async_copy(
                    table_hbm.at[idx_v.at[pl.ds(r * SEQ, G0)]],
                    rows_v.at[pl.ds(r * SEQ, G0)],
                    sem_g,
                )
                gb = pltpu.async_copy(
                    table_hbm.at[idx_v.at[pl.ds(r * SEQ + G0, G1)]],
                    rows_v.at[pl.ds(r * SEQ + G0, G1)],
                    sem_g,
                )
                ga.wait()
                gb.wait()

    def fire_outs(rows_v, row0):
        return [
            pltpu.async_copy(
                rows_v.at[pl.ds(r * SEQ, SEQ), :],
                out_hbm.at[pl.ds((row0 + r) * SEQ, SEQ), :],
                sem_o,
            )
            for r in range(CH)
        ]

    def refill_slow_rows(rows_v, counts):
        for r in range(CH):
            @pl.when(counts[r] != SEQ)
            def _refill():
                pltpu.async_copy(
                    tab_rows, rows_v.at[pl.ds(r * SEQ, SEQ), :], sem_g
                ).wait()

    def pair_body(i, _):
        row0 = base_row + i * 2 * CH
        pltpu.sync_copy(story_hbm.at[pl.ds(row0, 2 * CH), :], story_v)

        counts0 = [_positions_row(story_v, idx_bufs[0], r, r)
                   for r in range(CH)]
        gather_slow_rows(idx_bufs[0], rows_bufs[0], counts0)
        o0 = fire_outs(rows_bufs[0], row0)

        counts1 = [_positions_row(story_v, idx_bufs[1], CH + r, r)
                   for r in range(CH)]
        gather_slow_rows(idx_bufs[1], rows_bufs[1], counts1)
        o1 = fire_outs(rows_bufs[1], row0 + CH)

        for cp in o0:
            cp.wait()
        refill_slow_rows(rows_bufs[0], counts0)
        for cp in o1:
            cp.wait()
        refill_slow_rows(rows_bufs[1], counts1)
        return ()

    lax.fori_loop(0, NPAIRS, pair_body, ())


@jax.jit
def _encode(story2, table):
    mesh = plsc.VectorSubcoreMesh(core_axis_name="c", subcore_axis_name="s")
    f = functools.partial(
        pl.kernel,
        mesh=mesh,
        out_type=jax.ShapeDtypeStruct((BATCH * SEQ, EMB), jnp.float32),
        scratch_types=[
            pltpu.VMEM((2 * CH, SEQ), jnp.int32),
            pltpu.VMEM((NIDX,), jnp.int32),
            pltpu.VMEM((NIDX,), jnp.int32),
            pltpu.VMEM((NIDX, EMB), jnp.float32),
            pltpu.VMEM((NIDX, EMB), jnp.float32),
            pltpu.SemaphoreType.DMA,
            pltpu.SemaphoreType.DMA,
        ],
        compiler_params=pltpu.CompilerParams(
            needs_layout_passes=False, use_tc_tiling_on_sc=False
        ),
    )(_body)
    return f(story2, table)


def kernel(story, table):
    out = _encode(story[:, :, 0], table)
    return out.reshape(BATCH, SEQ, EMB)


# speedup vs baseline: 4.0473x; 1.0071x over previous
import functools

import jax
import jax.numpy as jnp
from jax import lax
from jax.experimental import pallas as pl
from jax.experimental.pallas import tpu as pltpu
from jax.experimental.pallas import tpu_sc as plsc

EMB = 64
SEQ = 200
BATCH = 16384
LANES = 16
CH = 4
NIDX = CH * SEQ
G0 = 104
G1 = SEQ - G0
NC = 2
NS = 16
NW = NC * NS
ROWS_PER_W = BATCH // NW
NPAIRS = ROWS_PER_W // (2 * CH)
NFULL = SEQ // LANES
TAIL_OFF = SEQ - LANES


def _positions_row(story_v, idx_v, sr, r):
    lane = lax.iota(jnp.int32, 16)
    low8 = lane < 8
    zeros = jnp.zeros(LANES, jnp.int32)
    carry = zeros
    carry183 = zeros
    for j in range(NFULL):
        tok = story_v[sr, pl.ds(j * LANES, LANES)]
        m = tok != 0
        ones = jnp.where(m, jnp.int32(1), jnp.int32(0))
        csum = plsc.cumsum(ones)
        idx_v[pl.ds(r * SEQ + j * LANES, LANES)] = jnp.where(
            m, csum + carry, jnp.int32(0)
        )
        if j == NFULL - 1:
            carry183 = carry + plsc.all_reduce_population_count(
                jnp.logical_and(m, low8))
        carry = carry + plsc.all_reduce_population_count(m)
    tok = story_v[sr, pl.ds(TAIL_OFF, LANES)]
    m = tok != 0
    ones = jnp.where(m, jnp.int32(1), jnp.int32(0))
    csum = plsc.cumsum(ones)
    idx_v[pl.ds(r * SEQ + TAIL_OFF, LANES)] = jnp.where(
        m, csum + carry183, jnp.int32(0)
    )
    total = carry183 + plsc.all_reduce_population_count(m)
    return lax.reduce_max(total, axes=(0,))


def _body(story_hbm, table_hbm, out_hbm, story_v, idx0_v, idx1_v, rows0_v,
          rows1_v, sem_g, sem_o):
    wid = lax.axis_index("s") * NC + lax.axis_index("c")
    base_row = wid * ROWS_PER_W

    idx_bufs = (idx0_v, idx1_v)
    rows_bufs = (rows0_v, rows1_v)
    tab_rows = table_hbm.at[pl.ds(1, SEQ), :]

    fills = [
        pltpu.async_copy(tab_rows, rows.at[pl.ds(r * SEQ, SEQ), :], sem_o)
        for rows in rows_bufs
        for r in range(CH)
    ]
    for cp in fills:
        cp.wait()

    def gather_slow_rows(idx_v, rows_v, counts):
        pend = []
        for r in range(CH):
            cond = counts[r] != SEQ

            @pl.when(cond)
            def _fire():
                pend.append((cond, pltpu.async_copy(
                    table_hbm.at[idx_v.at[pl.ds(r * SEQ, G0)]],
                    rows_v.at[pl.ds(r * SEQ, G0)],
                    sem_g,
                )))
                pend.append((cond, pltpu.async_copy(
                    table_hbm.at[idx_v.at[pl.ds(r * SEQ + G0, G1)]],
                    rows_v.at[pl.ds(r * SEQ + G0, G1)],
                    sem_g,
                )))

        for cond, cp in pend:
            @pl.when(cond)
            def _wait():
                cp.wait()

    def fire_outs(rows_v, row0):
        return [
            pltpu.async_copy(
                rows_v.at[pl.ds(r * SEQ, SEQ), :],
                out_hbm.at[pl.ds((row0 + r) * SEQ, SEQ), :],
                sem_o,
            )
            for r in range(CH)
        ]

    def fire_refills(rows_v, counts):
        pend = []
        for r in range(CH):
            cond = counts[r] != SEQ

            @pl.when(cond)
            def _refill():
                pend.append((cond, pltpu.async_copy(
                    tab_rows, rows_v.at[pl.ds(r * SEQ, SEQ), :], sem_g
                )))
        return pend

    def wait_refills(pend):
        for cond, cp in pend:
            @pl.when(cond)
            def _wait():
                cp.wait()

    def pair_body(i, _):
        row0 = base_row + i * 2 * CH
        pltpu.sync_copy(story_hbm.at[pl.ds(row0, 2 * CH), :], story_v)

        counts0 = [_positions_row(story_v, idx_bufs[0], r, r)
                   for r in range(CH)]
        gather_slow_rows(idx_bufs[0], rows_bufs[0], counts0)
        o0 = fire_outs(rows_bufs[0], row0)

        counts1 = [_positions_row(story_v, idx_bufs[1], CH + r, r)
                   for r in range(CH)]
        gather_slow_rows(idx_bufs[1], rows_bufs[1], counts1)
        o1 = fire_outs(rows_bufs[1], row0 + CH)

        for cp in o0:
            cp.wait()
        rf0 = fire_refills(rows_bufs[0], counts0)
        for cp in o1:
            cp.wait()
        rf1 = fire_refills(rows_bufs[1], counts1)
        wait_refills(rf0)
        wait_refills(rf1)
        return ()

    lax.fori_loop(0, NPAIRS, pair_body, ())


@jax.jit
def _encode(story2, table):
    mesh = plsc.VectorSubcoreMesh(core_axis_name="c", subcore_axis_name="s")
    f = functools.partial(
        pl.kernel,
        mesh=mesh,
        out_type=jax.ShapeDtypeStruct((BATCH * SEQ, EMB), jnp.float32),
        scratch_types=[
            pltpu.VMEM((2 * CH, SEQ), jnp.int32),
            pltpu.VMEM((NIDX,), jnp.int32),
            pltpu.VMEM((NIDX,), jnp.int32),
            pltpu.VMEM((NIDX, EMB), jnp.float32),
            pltpu.VMEM((NIDX, EMB), jnp.float32),
            pltpu.SemaphoreType.DMA,
            pltpu.SemaphoreType.DMA,
        ],
        compiler_params=pltpu.CompilerParams(
            needs_layout_passes=False, use_tc_tiling_on_sc=False
        ),
    )(_body)
    return f(story2, table)


def kernel(story, table):
    out = _encode(story[:, :, 0], table)
    return out.reshape(BATCH, SEQ, EMB)
